# baseline (speedup 1.0000x reference)
import functools

import jax
import jax.numpy as jnp
from jax import lax
from jax.experimental import pallas as pl
from jax.experimental.pallas import tpu as pltpu
from jax.experimental.pallas import tpu_sc as plsc

NUM_BUCKETS = 1000000
B = 4096
L = 200
D = 32
C = 100
NC = 2
NS = 16
NW = NC * NS
B_PER_W = B // NW
LANES = 16
S = 262144
TBLK = 8192
XROWS = 2 * B


def _tc_relayout(tableT, xT):
    max_blk = (NUM_BUCKETS - 1) // TBLK
    nsteps = S // TBLK
    xcols = B // (nsteps // 2)

    def body(t0, t1, t2, t3, x_ref, o_ref, o2_ref):
        m = jnp.concatenate(
            [t0[...], t1[...], t2[...], t3[...]], axis=0)
        o_ref[...] = m.T
        xt = x_ref[...].T
        o2_ref[...] = ((xt & (S - 1)) << 2) | lax.shift_right_logical(xt, 18)

    def in_map(i, k):
        return (0, jnp.minimum((S // TBLK) * k + i, max_blk))

    return pl.pallas_call(
        body,
        grid=(nsteps,),
        in_specs=[
            pl.BlockSpec((D, TBLK), functools.partial(in_map, k=k))
            for k in range(4)
        ] + [
            pl.BlockSpec((128, xcols),
                         lambda i: (i // (nsteps // 2), i % (nsteps // 2))),
        ],
        out_specs=[
            pl.BlockSpec((TBLK, 128), lambda i: (i, 0)),
            pl.BlockSpec((xcols, 128), lambda i: (i, 0)),
        ],
        out_shape=[
            jax.ShapeDtypeStruct((S, 128), jnp.float32),
            jax.ShapeDtypeStruct((XROWS, 128), jnp.int32),
        ],
    )(tableT, tableT, tableT, tableT, xT)


def _sc_sum_embeddings(x_lin, table):
    mesh = plsc.VectorSubcoreMesh(core_axis_name="c", subcore_axis_name="s")
    NBUF = 8
    TAIL = L - 128

    @functools.partial(
        pl.kernel,
        mesh=mesh,
        out_type=jax.ShapeDtypeStruct((B, D), jnp.float32),
        compiler_params=pltpu.CompilerParams(use_tc_tiling_on_sc=False),
        scratch_types=[
            pltpu.VMEM((2, B_PER_W, 128), jnp.int32),
            pltpu.VMEM((NBUF, L, D), jnp.float32),
            pltpu.VMEM((B_PER_W, D), jnp.float32),
            pltpu.SemaphoreType.DMA((NBUF,)),
        ],
    )
    def sc_kernel(x_hbm, table_hbm, out_hbm, idx_v, rows_v, acc_v, sems):
        wid = lax.axis_index("s") * NC + lax.axis_index("c")
        base = wid * B_PER_W
        pltpu.sync_copy(x_hbm.at[pl.ds(base, B_PER_W)], idx_v.at[0])
        pltpu.sync_copy(x_hbm.at[pl.ds(B + base, B_PER_W)], idx_v.at[1])

        zero = jnp.zeros((LANES,), jnp.float32)

        def fire(r, p):
            pltpu.async_copy(
                table_hbm.at[idx_v.at[0, r]],
                rows_v.at[p, pl.ds(0, 128)], sems.at[p])
            pltpu.async_copy(
                table_hbm.at[idx_v.at[1, r, pl.ds(0, TAIL)]],
                rows_v.at[p, pl.ds(128, TAIL)], sems.at[p])

        def drain(p):
            pltpu.make_async_copy(
                table_hbm.at[pl.ds(0, L)], rows_v.at[p], sems.at[p]).wait()

        def accumulate(r, p):
            def red(jo, acc):
                a00, a01, a10, a11 = acc
                j = jo * 10
                for u in range(0, 10, 2):
                    a00 = a00 + rows_v[p, j + u, pl.ds(0, LANES)]
                    a10 = a10 + rows_v[p, j + u, pl.ds(LANES, LANES)]
                    a01 = a01 + rows_v[p, j + u + 1, pl.ds(0, LANES)]
                    a11 = a11 + rows_v[p, j + u + 1, pl.ds(LANES, LANES)]
                return a00, a01, a10, a11

            a00, a01, a10, a11 = lax.fori_loop(
                0, L // 10, red, (zero, zero, zero, zero))
            acc_v[r, pl.ds(0, LANES)] = a00 + a01
            acc_v[r, pl.ds(LANES, LANES)] = a10 + a11

        for p in range(NBUF):
            fire(p, p)

        def group_body(g, carry):
            r0 = g * NBUF
            for p in range(NBUF):
                drain(p)
                accumulate(r0 + p, p)
                fire(r0 + p + NBUF, p)
            return carry

        lax.fori_loop(0, B_PER_W // NBUF - 1, group_body, 0)

        for p in range(NBUF):
            drain(p)
            accumulate(B_PER_W - NBUF + p, p)

        pltpu.sync_copy(acc_v, out_hbm.at[pl.ds(base, B_PER_W)])

    return sc_kernel(x_lin, table)


def _tc_head(sums, W, b2):
    def body(s_ref, w_ref, b_ref, o_ref):
        o_ref[...] = (
            jnp.dot(s_ref[...] * (1.0 / L), w_ref[...],
                    preferred_element_type=jnp.float32)
            + b_ref[...]
        )

    return pl.pallas_call(
        body,
        out_shape=jax.ShapeDtypeStruct((B, C), jnp.float32),
    )(sums, W, b2)


@jax.jit
def kernel(x, table, W, b):
    table_lin, x_lin = _tc_relayout(table.T, x.T)
    sums = _sc_sum_embeddings(x_lin, table_lin.reshape(4 * S, D))
    return _tc_head(sums, W, b.reshape(1, C))


# speedup vs baseline: 1.0268x; 1.0268x over previous
import functools

import jax
import jax.numpy as jnp
from jax import lax
from jax.experimental import pallas as pl
from jax.experimental.pallas import tpu as pltpu
from jax.experimental.pallas import tpu_sc as plsc

NUM_BUCKETS = 1000000
B = 4096
L = 200
D = 32
C = 100
NC = 2
NS = 16
NW = NC * NS
B_PER_W = B // NW
LANES = 16
S = 262144
TBLK = 8192
XROWS = 2 * B


def _tc_relayout(tableT, xT):
    max_blk = (NUM_BUCKETS - 1) // TBLK
    nsteps = S // TBLK
    xcols = B // (nsteps // 2)

    def body(t0, t1, t2, t3, x_ref, o_ref, o2_ref):
        m = jnp.concatenate(
            [t0[...], t1[...], t2[...], t3[...]], axis=0)
        o_ref[...] = m.T
        xt = x_ref[...].T
        o2_ref[...] = ((xt & (S - 1)) << 2) | lax.shift_right_logical(xt, 18)

    def in_map(i, k):
        return (0, jnp.minimum((S // TBLK) * k + i, max_blk))

    return pl.pallas_call(
        body,
        grid=(nsteps,),
        in_specs=[
            pl.BlockSpec((D, TBLK), functools.partial(in_map, k=k))
            for k in range(4)
        ] + [
            pl.BlockSpec((128, xcols),
                         lambda i: (i // (nsteps // 2), i % (nsteps // 2))),
        ],
        out_specs=[
            pl.BlockSpec((TBLK, 128), lambda i: (i, 0)),
            pl.BlockSpec((xcols, 128), lambda i: (i, 0)),
        ],
        out_shape=[
            jax.ShapeDtypeStruct((S, 128), jnp.float32),
            jax.ShapeDtypeStruct((XROWS, 128), jnp.int32),
        ],
    )(tableT, tableT, tableT, tableT, xT)


def _sc_sum_embeddings(x_lin, table):
    mesh = plsc.VectorSubcoreMesh(core_axis_name="c", subcore_axis_name="s")
    NBUF = 8
    TAIL = L - 128

    @functools.partial(
        pl.kernel,
        mesh=mesh,
        out_type=jax.ShapeDtypeStruct((B, D), jnp.float32),
        compiler_params=pltpu.CompilerParams(use_tc_tiling_on_sc=False),
        scratch_types=[
            pltpu.VMEM((2, B_PER_W, 128), jnp.int32),
            pltpu.VMEM((NBUF, L, D), jnp.float32),
            pltpu.VMEM((B_PER_W, D), jnp.float32),
            pltpu.SemaphoreType.DMA((NBUF,)),
        ],
    )
    def sc_kernel(x_hbm, table_hbm, out_hbm, idx_v, rows_v, acc_v, sems):
        wid = lax.axis_index("s") * NC + lax.axis_index("c")
        base = wid * B_PER_W
        pltpu.sync_copy(x_hbm.at[pl.ds(base, B_PER_W)], idx_v.at[0])
        pltpu.sync_copy(x_hbm.at[pl.ds(B + base, B_PER_W)], idx_v.at[1])

        zero = jnp.zeros((LANES,), jnp.float32)

        def fire(r, p):
            pltpu.async_copy(
                table_hbm.at[idx_v.at[0, r]],
                rows_v.at[p, pl.ds(0, 128)], sems.at[p])
            pltpu.async_copy(
                table_hbm.at[idx_v.at[1, r, pl.ds(0, TAIL)]],
                rows_v.at[p, pl.ds(128, TAIL)], sems.at[p])

        def drain(p):
            pltpu.make_async_copy(
                table_hbm.at[pl.ds(0, L)], rows_v.at[p], sems.at[p]).wait()

        def accumulate(r, p):
            def red(jo, acc):
                a00, a01, a10, a11 = acc
                j = jo * 10
                for u in range(0, 10, 2):
                    a00 = a00 + rows_v[p, j + u, pl.ds(0, LANES)]
                    a10 = a10 + rows_v[p, j + u, pl.ds(LANES, LANES)]
                    a01 = a01 + rows_v[p, j + u + 1, pl.ds(0, LANES)]
                    a11 = a11 + rows_v[p, j + u + 1, pl.ds(LANES, LANES)]
                return a00, a01, a10, a11

            a00, a01, a10, a11 = lax.fori_loop(
                0, L // 10, red, (zero, zero, zero, zero))
            acc_v[r, pl.ds(0, LANES)] = a00 + a01
            acc_v[r, pl.ds(LANES, LANES)] = a10 + a11

        for p in range(NBUF):
            fire(p, p)

        def group_body(g, carry):
            r0 = g * NBUF
            for p in range(NBUF):
                drain(p)
                accumulate(r0 + p, p)
                fire(r0 + p + NBUF, p)
            return carry

        lax.fori_loop(0, B_PER_W // NBUF - 1, group_body, 0)

        for p in range(NBUF):
            drain(p)
            accumulate(B_PER_W - NBUF + p, p)

        pltpu.sync_copy(acc_v, out_hbm.at[pl.ds(base, B_PER_W)])

    return sc_kernel(x_lin, table)


def _tc_head(sums, W, b2):
    def body(s_ref, w_ref, b_ref, o_ref):
        o_ref[...] = (
            lax.dot_general(
                w_ref[...] * (1.0 / L), s_ref[...],
                dimension_numbers=(((0,), (1,)), ((), ())),
                preferred_element_type=jnp.float32)
            + b_ref[...]
        )

    return pl.pallas_call(
        body,
        out_shape=jax.ShapeDtypeStruct((C, B), jnp.float32),
    )(sums, W, b2)


@jax.jit
def kernel(x, table, W, b):
    table_lin, x_lin = _tc_relayout(table.T, x.T)
    sums = _sc_sum_embeddings(x_lin, table_lin.reshape(4 * S, D))
    return _tc_head(sums, W, b.reshape(C, 1)).T
